# Initial kernel scaffold; baseline (speedup 1.0000x reference)
import jax
import jax.numpy as jnp
from jax.experimental import pallas as pl


def kernel(idxs, table, W1, b1, W2, b2):
    raise NotImplementedError("write your pallas kernel here")


# speedup vs baseline: 13.4030x; 13.4030x over previous
import functools

import jax
import jax.numpy as jnp
from jax import lax
from jax.experimental import pallas as pl
from jax.experimental.pallas import tpu as pltpu
from jax.experimental.pallas import tpu_sc as plsc

VOCAB = 100000
FRONT = 256
EMBED = 64

_NC = 2
_NS = 16
_NW = _NC * _NS


def _table_transform(table, W1, b1, W2, b2):
    BM = 2000
    grid = (VOCAB // BM,)

    def body(x_ref, w1_ref, b1_ref, w2_ref, b2_ref, o_ref):
        x = x_ref[:]
        h = lax.dot_general(x, w1_ref[:], (((1,), (1,)), ((), ())),
                            preferred_element_type=jnp.float32) + b1_ref[:]
        g = h * 0.5 * (1.0 + lax.erf(h * (2.0 ** -0.5)))
        o_ref[:] = lax.dot_general(g, w2_ref[:], (((1,), (1,)), ((), ())),
                                   preferred_element_type=jnp.float32) + b2_ref[:]

    return pl.pallas_call(
        body,
        grid=grid,
        in_specs=[
            pl.BlockSpec((BM, FRONT), lambda i: (i, 0)),
            pl.BlockSpec((EMBED, FRONT), lambda i: (0, 0)),
            pl.BlockSpec((1, EMBED), lambda i: (0, 0)),
            pl.BlockSpec((EMBED, EMBED), lambda i: (0, 0)),
            pl.BlockSpec((1, EMBED), lambda i: (0, 0)),
        ],
        out_specs=pl.BlockSpec((BM, EMBED), lambda i: (i, 0)),
        out_shape=jax.ShapeDtypeStruct((VOCAB, EMBED), jnp.float32),
    )(table, W1, b1.reshape(1, EMBED), W2, b2.reshape(1, EMBED))


def _sc_gather(t2, idx_flat):
    total = idx_flat.shape[0]
    b_per_w = total // _NW
    C = 1280
    n_chunks = b_per_w // C

    mesh = plsc.VectorSubcoreMesh(core_axis_name="c", subcore_axis_name="s")

    @functools.partial(
        pl.kernel,
        mesh=mesh,
        out_type=jax.ShapeDtypeStruct((total, EMBED), jnp.float32),
        scratch_types=[
            pltpu.VMEM((C,), jnp.int32),
            pltpu.VMEM((C, EMBED), jnp.float32),
            pltpu.SemaphoreType.DMA,
        ],
        compiler_params=pltpu.CompilerParams(use_tc_tiling_on_sc=False),
    )
    def k(t2_hbm, idx_hbm, out_hbm, idx_v, rows_v, sem):
        wid = lax.axis_index("s") * _NC + lax.axis_index("c")
        base = wid * b_per_w
        for c in range(n_chunks):
            off = base + c * C
            pltpu.sync_copy(idx_hbm.at[pl.ds(off, C)], idx_v)
            pltpu.async_copy(t2_hbm.at[idx_v], rows_v, sem).wait()
            pltpu.sync_copy(rows_v, out_hbm.at[pl.ds(off, C)])

    return k(t2, idx_flat)


def kernel(idxs, table, W1, b1, W2, b2):
    B, L = idxs.shape
    t2 = _table_transform(table, W1, b1, W2, b2)
    idx_flat = idxs.reshape(-1).astype(jnp.int32)
    out = _sc_gather(t2, idx_flat)
    return out.reshape(B, L, EMBED)
